# baseline (speedup 1.0000x reference)
import functools

import jax
import jax.numpy as jnp
from jax import lax
from jax.experimental import pallas as pl
from jax.experimental.pallas import tpu as pltpu
from jax.experimental.pallas import tpu_sc as plsc

_H = 32
_W = 32
_NH = 16
_HW = _H * _W
_D = 2 * _W - 1
_NC = 2
_NS = 16
_NW = _NC * _NS
_PW = 1920


def _phase_body(th_ref, ph_ref, bank_scr):
    r = lax.broadcasted_iota(jnp.int32, (_D, _D), 0)
    c = lax.broadcasted_iota(jnp.int32, (_D, _D), 1)
    perm = (r + c == _D - 1).astype(jnp.float32)
    xf = jax.lax.dot(
        perm,
        jax.lax.dot(th_ref[0], perm, precision=jax.lax.Precision.HIGHEST),
        precision=jax.lax.Precision.HIGHEST,
    )
    for w in range(_W):
        bank_scr[w] = xf[:, 31 - w : 63 - w]
    x = bank_scr[...].reshape(_W, _D * _W)
    for p in range(4):
        ph_ref[p, 0] = x[:, 32 * p : 32 * p + _PW]


def _build_phases(th):
    return pl.pallas_call(
        _phase_body,
        grid=(_NH,),
        in_specs=[pl.BlockSpec((1, _D, _D), lambda h: (h, 0, 0))],
        out_specs=pl.BlockSpec((4, 1, _W, _PW), lambda h: (0, h, 0, 0)),
        out_shape=jax.ShapeDtypeStruct((4, _NH, _W, _PW), jnp.float32),
        scratch_shapes=[pltpu.VMEM((_W, _D, _W), jnp.float32)],
    )(th)


_sc_mesh = plsc.VectorSubcoreMesh(core_axis_name="c", subcore_axis_name="s")


@functools.partial(
    pl.kernel,
    mesh=_sc_mesh,
    out_type=jax.ShapeDtypeStruct((1, _NH, _HW, _HW), jnp.float32),
    scratch_types=[
        pltpu.VMEM((_H, _PW), jnp.float32),
        pltpu.VMEM((_H, _PW), jnp.float32),
        pltpu.SemaphoreType.DMA,
        pltpu.SemaphoreType.DMA,
        pltpu.SemaphoreType.DMA,
        pltpu.SemaphoreType.DMA,
    ],
    compiler_params=pltpu.CompilerParams(use_tc_tiling_on_sc=True),
)
def _stage_b(ph_hbm, out_hbm, buf0, buf1, isem0, isem1, osem0, osem1):
    wid = lax.axis_index("s") * _NC + lax.axis_index("c")
    h = wid // 2
    hi_base = (wid % 2) * 16
    bufs = (buf0, buf1)
    isems = (isem0, isem1)
    osems = (osem0, osem1)

    groups = [(p, [t for t in range(16) if (3 - t) % 4 == p]) for p in range(4)]

    def read_phase(gi, b):
        p = groups[gi][0]
        return pltpu.async_copy(ph_hbm.at[p, h, :, :], bufs[b], isems[b])

    ins = {0: read_phase(0, 0)}
    pending = {0: [], 1: []}
    for gi in range(4):
        b = gi & 1
        nb = b ^ 1
        if gi + 1 < 4:
            for hnd in pending[nb]:
                hnd.wait()
            pending[nb] = []
            ins[gi + 1] = read_phase(gi + 1, nb)
        ins[gi].wait()
        writes = []
        for t in groups[gi][1]:
            hi = hi_base + t
            q = (31 - hi) // 4
            writes.append(
                pltpu.async_copy(
                    bufs[b].at[:, pl.ds(q * 128, _HW)],
                    out_hbm.at[0, h, pl.ds(hi * _W, _W), :],
                    osems[b],
                )
            )
        pending[b] = writes
    for hnd in pending[0] + pending[1]:
        hnd.wait()


def kernel(relative_bias_table, relative_index):
    del relative_index
    th = relative_bias_table.T.reshape(_NH, _D, _D)
    phases = _build_phases(th)
    return _stage_b(phases)


# speedup vs baseline: 1.0692x; 1.0692x over previous
import functools

import jax
import jax.numpy as jnp
from jax import lax
from jax.experimental import pallas as pl
from jax.experimental.pallas import tpu as pltpu
from jax.experimental.pallas import tpu_sc as plsc

_H = 32
_W = 32
_NH = 16
_HW = _H * _W
_D = 2 * _W - 1
_NC = 2
_NS = 16
_NW = _NC * _NS
_PW = 1920


def _phase_body(th_ref, ph_ref, bank_scr):
    r = lax.broadcasted_iota(jnp.int32, (_D, _D), 0)
    c = lax.broadcasted_iota(jnp.int32, (_D, _D), 1)
    perm = (r + c == _D - 1).astype(jnp.float32)
    for i in range(4):
        xf = jax.lax.dot(
            perm,
            jax.lax.dot(th_ref[i], perm, precision=jax.lax.Precision.HIGHEST),
            precision=jax.lax.Precision.HIGHEST,
        )
        for w in range(_W):
            bank_scr[i, w] = xf[:, 31 - w : 63 - w]
    x = bank_scr[...].reshape(128, _D * _W)
    for p in range(4):
        ph_ref[p] = x[:, 32 * p : 32 * p + _PW]


def _build_phases(th):
    return pl.pallas_call(
        _phase_body,
        grid=(4,),
        in_specs=[pl.BlockSpec((4, _D, _D), lambda r: (r, 0, 0))],
        out_specs=pl.BlockSpec((4, 128, _PW), lambda r: (0, r, 0)),
        out_shape=jax.ShapeDtypeStruct((4, _NH * _W, _PW), jnp.float32),
        scratch_shapes=[pltpu.VMEM((4, _W, _D, _W), jnp.float32)],
    )(th)


_sc_mesh = plsc.VectorSubcoreMesh(core_axis_name="c", subcore_axis_name="s")


@functools.partial(
    pl.kernel,
    mesh=_sc_mesh,
    out_type=jax.ShapeDtypeStruct((1, _NH, _HW, _HW), jnp.float32),
    scratch_types=[
        pltpu.VMEM((_H, _PW), jnp.float32),
        pltpu.VMEM((_H, _PW), jnp.float32),
        pltpu.SemaphoreType.DMA,
        pltpu.SemaphoreType.DMA,
        pltpu.SemaphoreType.DMA,
        pltpu.SemaphoreType.DMA,
    ],
    compiler_params=pltpu.CompilerParams(use_tc_tiling_on_sc=True),
)
def _stage_b(ph_hbm, out_hbm, buf0, buf1, isem0, isem1, osem0, osem1):
    wid = lax.axis_index("s") * _NC + lax.axis_index("c")
    h = wid // 2
    hi_base = (wid % 2) * 16
    bufs = (buf0, buf1)
    isems = (isem0, isem1)
    osems = (osem0, osem1)

    groups = [(p, [t for t in range(16) if (3 - t) % 4 == p]) for p in range(4)]

    def read_phase(gi, b):
        p = groups[gi][0]
        return pltpu.async_copy(
            ph_hbm.at[p, pl.ds(h * _W, _W), :], bufs[b], isems[b]
        )

    ins = {0: read_phase(0, 0)}
    pending = {0: [], 1: []}
    for gi in range(4):
        b = gi & 1
        nb = b ^ 1
        if gi + 1 < 4:
            for hnd in pending[nb]:
                hnd.wait()
            pending[nb] = []
            ins[gi + 1] = read_phase(gi + 1, nb)
        ins[gi].wait()
        writes = []
        for t in groups[gi][1]:
            hi = hi_base + t
            q = (31 - hi) // 4
            writes.append(
                pltpu.async_copy(
                    bufs[b].at[:, pl.ds(q * 128, _HW)],
                    out_hbm.at[0, h, pl.ds(hi * _W, _W), :],
                    osems[b],
                )
            )
        pending[b] = writes
    for hnd in pending[0] + pending[1]:
        hnd.wait()


def kernel(relative_bias_table, relative_index):
    del relative_index
    th = relative_bias_table.T.reshape(_NH, _D, _D)
    phases = _build_phases(th)
    return _stage_b(phases)
